# Initial kernel scaffold; baseline (speedup 1.0000x reference)
import jax
import jax.numpy as jnp
from jax.experimental import pallas as pl


def kernel(x, edge_index, W0, g0, b0, W1, g1, b1, W2, g2, b2, Wm0, bm0, Wm1, bm1):
    raise NotImplementedError("write your pallas kernel here")


# speedup vs baseline: 3.0686x; 3.0686x over previous
import functools

import jax
import jax.numpy as jnp
from jax import lax
from jax.experimental import pallas as pl
from jax.experimental.pallas import tpu as pltpu
from jax.experimental.pallas import tpu_sc as plsc

N_NODES = 10000
D = 128
N_EDGES = 320000

NT = 16
CHUNK = 128
NCHUNK = 160
EDGES_PAD = NT * NCHUNK * CHUNK
NB = 2
PASSES = 4
PCHUNK = NCHUNK // PASSES
N_PAD = 10240
DUMMY_ROW = 10016
ROWS_PER_TILE = N_PAD // NT


def _segment_sum_sc(hw, src3, dst3, zeros):
    mesh = plsc.VectorSubcoreMesh(
        core_axis_name="c", subcore_axis_name="s", num_cores=1)

    @functools.partial(
        pl.kernel,
        out_type=jax.ShapeDtypeStruct((N_PAD, D), jnp.float32),
        mesh=mesh,
        scratch_types=[
            pltpu.VMEM((PCHUNK, CHUNK), jnp.int32),
            pltpu.VMEM((PCHUNK, CHUNK), jnp.int32),
            [pltpu.VMEM((CHUNK, D), jnp.float32) for _ in range(NB)],
            pltpu.VMEM_SHARED((N_PAD, D), jnp.float32),
            [pltpu.SemaphoreType.DMA for _ in range(NB)],
            [pltpu.SemaphoreType.DMA for _ in range(NB)],
        ],
    )
    def k(hw_hbm, src_hbm, dst_hbm, zero_hbm, out_hbm,
          src_v, dst_v, bufs, acc, gsems, ssems):
        s = lax.axis_index("s")

        pltpu.sync_copy(
            zero_hbm.at[pl.ds(s * ROWS_PER_TILE, ROWS_PER_TILE)],
            acc.at[pl.ds(s * ROWS_PER_TILE, ROWS_PER_TILE)])
        plsc.subcore_barrier()

        for p in range(PASSES):
            pltpu.sync_copy(src_hbm.at[s, pl.ds(p * PCHUNK, PCHUNK)], src_v)
            pltpu.sync_copy(dst_hbm.at[s, pl.ds(p * PCHUNK, PCHUNK)], dst_v)

            for b in range(NB):
                pltpu.async_copy(hw_hbm.at[src_v.at[b]], bufs[b], gsems[b])

            def body(i, _):
                j0 = i * NB
                for b in range(NB):
                    j = j0 + b
                    pltpu.make_async_copy(
                        hw_hbm.at[src_v.at[j]], bufs[b], gsems[b]).wait()
                    pltpu.async_copy(
                        bufs[b], acc.at[dst_v.at[j]], ssems[b], add=True)
                    pltpu.make_async_copy(
                        bufs[b], acc.at[dst_v.at[j]], ssems[b]).wait()

                    @pl.when(j + NB < PCHUNK)
                    def _():
                        pltpu.async_copy(
                            hw_hbm.at[src_v.at[j + NB]], bufs[b], gsems[b])

                return 0

            lax.fori_loop(0, PCHUNK // NB, body, 0)

        plsc.subcore_barrier()
        pltpu.sync_copy(
            acc.at[pl.ds(s * ROWS_PER_TILE, ROWS_PER_TILE)],
            out_hbm.at[pl.ds(s * ROWS_PER_TILE, ROWS_PER_TILE)],
        )

    return k(hw, src3, dst3, zeros)


def _mm_first(x, W):
    def body(x_ref, w_ref, o_ref):
        o_ref[...] = jnp.dot(x_ref[...], w_ref[...],
                             preferred_element_type=jnp.float32)

    return pl.pallas_call(
        body,
        out_shape=jax.ShapeDtypeStruct((N_NODES, D), jnp.float32),
    )(x, W)


def _bn_relu(p_ref, g_ref, b_ref):
    sarr = p_ref[pl.ds(0, N_NODES), :]
    mu = jnp.mean(sarr, axis=0, keepdims=True)
    d = sarr - mu
    var = jnp.mean(d * d, axis=0, keepdims=True)
    hn = g_ref[...] * d * lax.rsqrt(var + 1e-5) + b_ref[...]
    return jnp.maximum(hn, 0.0)


def _stage_mid(p, g, b, W):
    def body(p_ref, g_ref, b_ref, w_ref, o_ref):
        h = _bn_relu(p_ref, g_ref, b_ref)
        o_ref[...] = jnp.dot(h, w_ref[...],
                             preferred_element_type=jnp.float32)

    return pl.pallas_call(
        body,
        out_shape=jax.ShapeDtypeStruct((N_NODES, D), jnp.float32),
    )(p, g.reshape(1, D), b.reshape(1, D), W)


def _stage_final(p, g, b, Wm0, bm0, Wm1, bm1):
    def body(p_ref, g_ref, b_ref, w0_ref, b0_ref, w1_ref, b1_ref, o_ref):
        h = _bn_relu(p_ref, g_ref, b_ref)
        pooled = jnp.sum(h, axis=0, keepdims=True)
        z = jnp.maximum(
            jnp.dot(pooled, w0_ref[...],
                    preferred_element_type=jnp.float32) + b0_ref[...], 0.0)
        o_ref[...] = jnp.dot(z, w1_ref[...],
                             preferred_element_type=jnp.float32) + b1_ref[...]

    return pl.pallas_call(
        body,
        out_shape=jax.ShapeDtypeStruct((1, 16), jnp.float32),
    )(p, g.reshape(1, D), b.reshape(1, D),
      Wm0, bm0.reshape(1, -1), Wm1, bm1.reshape(1, -1))


@jax.jit
def kernel(x, edge_index, W0, g0, b0, W1, g1, b1, W2, g2, b2,
           Wm0, bm0, Wm1, bm1):
    pad = EDGES_PAD - N_EDGES
    src3 = jnp.concatenate(
        [edge_index[0].astype(jnp.int32), jnp.zeros((pad,), jnp.int32)]
    ).reshape(NT, NCHUNK, CHUNK)
    dst3 = jnp.concatenate(
        [edge_index[1].astype(jnp.int32),
         jnp.full((pad,), DUMMY_ROW, jnp.int32)]
    ).reshape(NT, NCHUNK, CHUNK)
    zeros = jnp.zeros((N_PAD, D), jnp.float32)

    hw = _mm_first(x, W0)
    p = _segment_sum_sc(hw, src3, dst3, zeros)
    hw = _stage_mid(p, g0, b0, W1)
    p = _segment_sum_sc(hw, src3, dst3, zeros)
    hw = _stage_mid(p, g1, b1, W2)
    p = _segment_sum_sc(hw, src3, dst3, zeros)
    return _stage_final(p, g2, b2, Wm0, bm0, Wm1, bm1)
